# baseline (speedup 1.0000x reference)
import functools
import numpy as np

import jax
import jax.numpy as jnp
from jax import lax
from jax.experimental import pallas as pl
from jax.experimental.pallas import tpu as pltpu
from jax.experimental.pallas import tpu_sc as plsc

EPS = 1e-5
NEG = -3.0e38
T = 256
W = 264
NGRAPH = 8
RN = 1000

_HI = jax.lax.Precision.HIGHEST
_dot = functools.partial(jnp.dot, preferred_element_type=jnp.float32,
                         precision=_HI)


def _dotb(a, b):
    return jax.lax.dot_general(
        a.astype(jnp.bfloat16), b,
        (((1,), (0,)), ((), ())), preferred_element_type=jnp.float32)


def _fold(lin, bn):
    inv = 1.0 / float(np.sqrt(1.0 + EPS))
    s = bn["gamma"] * inv
    return lin["W"] * s[None, :], (lin["b"] * s + bn["beta"])[None, :]


def _k3mm(v_ref, w_ref):
    acc = v_ref[:, 0:1] * w_ref[0:1, :]
    acc += v_ref[:, 1:2] * w_ref[1:2, :]
    acc += v_ref[:, 2:3] * w_ref[2:3, :]
    return acc


def _pre_body(x_ref, pos_ref, wx_ref, wp1_ref, wp2_ref, wp3_ref, b1_ref,
              a1_ref, bw1_ref, bw2_ref, bw3_ref):
    bw1 = _k3mm(pos_ref, wp1_ref)
    bw1_ref[...] = bw1
    bw2_ref[...] = _k3mm(pos_ref, wp2_ref)
    bw3_ref[...] = _k3mm(pos_ref, wp3_ref)
    a1_ref[...] = _k3mm(x_ref, wx_ref) + bw1 + b1_ref[...]


def _pre(x, pos, wx, wp1, wp2, wp3, b1):
    n = x.shape[0]
    nt = n // RN
    sh = jax.ShapeDtypeStruct((n, 64), jnp.float32)
    row = pl.BlockSpec((RN, 3), lambda i: (i, 0))
    w = pl.BlockSpec((3, 64), lambda i: (0, 0))
    return pl.pallas_call(
        _pre_body,
        grid=(nt,),
        in_specs=[row, row, w, w, w, w, pl.BlockSpec((1, 64), lambda i: (0, 0))],
        out_specs=tuple(pl.BlockSpec((RN, 64), lambda i: (i, 0))
                        for _ in range(4)),
        out_shape=(sh, sh, sh, sh),
    )(x, pos, wx, wp1, wp2, wp3, b1)


def _conv_body(n, e, cout, bases_ref, g_ref, g2_ref, dstr_ref, dstc_ref,
               w2_ref, b2_ref, w3_ref, b3_ref, out_ref):
    i = pl.program_id(0)

    @pl.when(i == 0)
    def _():
        out_ref[...] = jnp.full((n, cout), NEG, jnp.float32)

    base = pl.multiple_of(bases_ref[i], 8)
    d_row = dstr_ref[0]
    d_col = dstc_ref[0]
    j_row = d_row - base

    h = jnp.maximum(g_ref[:, :64] - g2_ref[:, 64:], 0.0)
    h = jnp.maximum(_dotb(h, w2_ref[...]) + b2_ref[...], 0.0)
    h = _dotb(h, w3_ref[...]) + b3_ref[...]

    e0 = i * T
    valid = (jax.lax.broadcasted_iota(jnp.int32, (T, 1), 0) + e0) < e
    h = jnp.where(valid, h, NEG).astype(jnp.bfloat16)
    negb = jnp.asarray(NEG, jnp.bfloat16)

    s = 1
    while s < T:
        hs = jnp.concatenate([jnp.full((s, cout), negb, jnp.bfloat16),
                              h[:-s]], 0)
        ds = jnp.concatenate([jnp.full((s, 1), -1, jnp.int32), d_col[:-s]], 0)
        h = jnp.where(d_col == ds, jnp.maximum(h, hs), h)
        s *= 2

    nd = jnp.concatenate([d_row[:, 1:], jnp.full((1, 1), -1, jnp.int32)], 1)
    isend = d_row != nd
    iota_w_t = jax.lax.broadcasted_iota(jnp.int32, (W, T), 0)
    pend = ((iota_w_t == j_row) & isend).astype(jnp.bfloat16)
    outw = lax.dot_general(pend, h, (((1,), (0,)), ((), ())),
                           preferred_element_type=jnp.float32)
    hasend = jnp.max(pend.astype(jnp.float32), axis=1, keepdims=True)
    outw = jnp.where(hasend > 0.0, outw, NEG)

    cur = out_ref[pl.ds(base, W), :]
    out_ref[pl.ds(base, W), :] = jnp.maximum(cur, outw)


def _conv(bases, g, g2, dstr, dstc, w2, b2, w3, b3, n, e, cout):
    nt = dstr.shape[0]
    grid_spec = pltpu.PrefetchScalarGridSpec(
        num_scalar_prefetch=1,
        grid=(nt,),
        in_specs=[
            pl.BlockSpec((T, 128), lambda i, b: (i, 0)),
            pl.BlockSpec((T, 128), lambda i, b: (i, 0)),
            pl.BlockSpec((1, 1, T), lambda i, b: (i, 0, 0)),
            pl.BlockSpec((1, T, 1), lambda i, b: (i, 0, 0)),
            pl.BlockSpec((64, 128), lambda i, b: (0, 0)),
            pl.BlockSpec((1, 128), lambda i, b: (0, 0)),
            pl.BlockSpec((128, cout), lambda i, b: (0, 0)),
            pl.BlockSpec((1, cout), lambda i, b: (0, 0)),
        ],
        out_specs=pl.BlockSpec((n, cout), lambda i, b: (0, 0)),
    )
    return pl.pallas_call(
        functools.partial(_conv_body, n, e, cout),
        grid_spec=grid_spec,
        out_shape=jax.ShapeDtypeStruct((n, cout), jnp.float32),
    )(bases, g, g2, dstr, dstc, w2, b2, w3, b3)


def _post_body(agg_ref, wg_ref, bg_ref, wx_ref, bwn_ref, b1n_ref, a_ref):
    t = jnp.maximum(_dotb(agg_ref[...], wg_ref[...]) + bg_ref[...], 0.0)
    a_ref[...] = _dotb(t, wx_ref[...]) + bwn_ref[...] + b1n_ref[...]


def _post(agg, wg, bg, wx, bwn, b1n):
    n, cout = agg.shape
    nt = n // RN
    return pl.pallas_call(
        _post_body,
        grid=(nt,),
        in_specs=[
            pl.BlockSpec((RN, cout), lambda i: (i, 0)),
            pl.BlockSpec((cout, cout), lambda i: (0, 0)),
            pl.BlockSpec((1, cout), lambda i: (0, 0)),
            pl.BlockSpec((cout, 64), lambda i: (0, 0)),
            pl.BlockSpec((RN, 64), lambda i: (i, 0)),
            pl.BlockSpec((1, 64), lambda i: (0, 0)),
        ],
        out_specs=pl.BlockSpec((RN, 64), lambda i: (i, 0)),
        out_shape=jax.ShapeDtypeStruct((n, 64), jnp.float32),
    )(agg, wg, bg, wx, bwn, b1n)


def _final_body(nt, agg_ref, batch_ref, wg_ref, bg_ref, w1_ref, b1_ref,
                w2_ref, b2_ref, w3_ref, b3_ref, out_ref, gacc):
    i = pl.program_id(0)

    @pl.when(i == 0)
    def _():
        gacc[...] = jnp.full(gacc.shape, NEG, jnp.float32)

    h = jnp.maximum(_dotb(agg_ref[...], wg_ref[...]) + bg_ref[...], 0.0)
    b = batch_ref[0]
    for gidx in range(NGRAPH):
        m = b == gidx
        mx = jnp.max(jnp.where(m, h, NEG), axis=0, keepdims=True)
        gacc[gidx:gidx + 1, :] = jnp.maximum(gacc[gidx:gidx + 1, :], mx)

    @pl.when(i == nt - 1)
    def _():
        gv = gacc[...]
        gv = jnp.where(gv >= 0.0, gv, 0.0)
        z = jnp.maximum(_dot(gv, w1_ref[...]) + b1_ref[...], 0.0)
        z = jnp.maximum(_dot(z, w2_ref[...]) + b2_ref[...], 0.0)
        out_ref[...] = _dot(z, w3_ref[...]) + b3_ref[...]


def _final(agg, batch3, wg, bg, w1, b1, w2, b2, w3, b3):
    n, cout = agg.shape
    nt = n // RN
    return pl.pallas_call(
        functools.partial(_final_body, nt),
        grid=(nt,),
        in_specs=[
            pl.BlockSpec((RN, cout), lambda i: (i, 0)),
            pl.BlockSpec((1, RN, 1), lambda i: (i, 0, 0)),
            pl.BlockSpec((cout, cout), lambda i: (0, 0)),
            pl.BlockSpec((1, cout), lambda i: (0, 0)),
            pl.BlockSpec((cout, 256), lambda i: (0, 0)),
            pl.BlockSpec((1, 256), lambda i: (0, 0)),
            pl.BlockSpec((256, 128), lambda i: (0, 0)),
            pl.BlockSpec((1, 128), lambda i: (0, 0)),
            pl.BlockSpec((128, 7), lambda i: (0, 0)),
            pl.BlockSpec((1, 7), lambda i: (0, 0)),
        ],
        out_specs=pl.BlockSpec((NGRAPH, 7), lambda i: (0, 0)),
        out_shape=jax.ShapeDtypeStruct((NGRAPH, 7), jnp.float32),
        scratch_shapes=[pltpu.VMEM((NGRAPH, cout), jnp.float32)],
    )(agg, batch3, wg, bg, w1, b1, w2, b2, w3, b3)


NW = 32
SC_CHUNK = 280


def _sc_gather2(table, idx_a, idx_b):
    ep = idx_a.shape[0]
    b_per_w = ep // NW
    nchunk = b_per_w // SC_CHUNK
    d = table.shape[1]
    mesh = plsc.VectorSubcoreMesh(core_axis_name="c", subcore_axis_name="s")
    sh = jax.ShapeDtypeStruct((ep, d), jnp.float32)

    @functools.partial(
        pl.kernel, mesh=mesh,
        out_type=(sh, sh),
        scratch_types=[
            pltpu.VMEM((SC_CHUNK,), jnp.int32),
            pltpu.VMEM((SC_CHUNK, d), jnp.float32),
            pltpu.VMEM((SC_CHUNK,), jnp.int32),
            pltpu.VMEM((SC_CHUNK, d), jnp.float32),
            pltpu.SemaphoreType.DMA,
            pltpu.SemaphoreType.DMA,
        ],
    )
    def gk(table_hbm, ia_hbm, ib_hbm, oa_hbm, ob_hbm,
           idx0, rows0, idx1, rows1, sem0, sem1):
        wid = lax.axis_index("s") * 2 + lax.axis_index("c")
        base = wid * b_per_w
        bufs = [(idx0, rows0, sem0), (idx1, rows1, sem1)]
        work = ([(ia_hbm, oa_hbm, c) for c in range(nchunk)]
                + [(ib_hbm, ob_hbm, c) for c in range(nchunk)])
        ih0, _, _ = work[0]
        pltpu.sync_copy(ih0.at[pl.ds(base, SC_CHUNK)], idx0)
        cp = pltpu.async_copy(table_hbm.at[idx0], rows0, sem0)
        for k in range(len(work)):
            if k + 1 < len(work):
                ihn, _, cn = work[k + 1]
                ivn, rvn, semn = bufs[(k + 1) % 2]
                pltpu.sync_copy(ihn.at[pl.ds(base + cn * SC_CHUNK,
                                             SC_CHUNK)], ivn)
                cpn = pltpu.async_copy(table_hbm.at[ivn], rvn, semn)
            cp.wait()
            _, oh, c = work[k]
            _, rv, _ = bufs[k % 2]
            pltpu.sync_copy(rv, oh.at[pl.ds(base + c * SC_CHUNK, SC_CHUNK)])
            if k + 1 < len(work):
                cp = cpn

    return gk(table, idx_a, idx_b)


def kernel(x, pos, edge_index, batch, params):
    n = x.shape[0]
    e = edge_index.shape[1] + n
    nt_e = (e + T - 1) // T
    epad = nt_e * T

    sl = jnp.arange(n, dtype=edge_index.dtype)
    src = jnp.concatenate([edge_index[0], sl])
    dst = jnp.concatenate([edge_index[1], sl])
    order = jnp.argsort(dst)
    sdst = jnp.concatenate(
        [dst[order], jnp.full((epad - e,), n - 1, jnp.int32)])
    ssrc = jnp.concatenate([src[order], jnp.zeros((epad - e,), jnp.int32)])
    bases = jnp.minimum((sdst.reshape(nt_e, T)[:, 0] // 8) * 8, n - W)
    sdstr = sdst.reshape(nt_e, 1, T)
    sdstc = sdst.reshape(nt_e, T, 1)
    batch3 = batch.reshape(n // RN, RN, 1)

    cps = [params["conv1"], params["conv2"], params["conv3"]]
    w1f = [_fold(p["l1"], p["bn1"]) for p in cps]
    w2f = [_fold(p["l2"], p["bn2"]) for p in cps]
    wgf = [_fold(p["g"], p["bng"]) for p in cps]
    cins = [3, 128, 256]
    couts = [128, 256, 512]
    wx = [w1f[l][0][:cins[l]] for l in range(3)]
    wp = [w1f[l][0][cins[l]:] for l in range(3)]
    b1 = [w1f[l][1] for l in range(3)]

    a, bw1, bw2, bw3 = _pre(x, pos, wx[0], wp[0], wp[1], wp[2], b1[0])
    bws = [bw1, bw2, bw3]

    for l in range(3):
        tab = jnp.concatenate([a, bws[l]], axis=1)
        g, g2 = _sc_gather2(tab, ssrc, sdst)
        agg = _conv(bases, g, g2, sdstr, sdstc,
                    w2f[l][0].astype(jnp.bfloat16), w2f[l][1],
                    cps[l]["l3"]["W"].astype(jnp.bfloat16),
                    cps[l]["l3"]["b"][None, :],
                    n, e, couts[l])
        if l < 2:
            a = _post(agg, wgf[l][0].astype(jnp.bfloat16), wgf[l][1],
                      wx[l + 1].astype(jnp.bfloat16), bws[l + 1], b1[l + 1])

    c1w, c1b = _fold(params["c1"], params["cbn1"])
    c2w, c2b = _fold(params["c2"], params["cbn2"])
    return _final(agg, batch3, wgf[2][0].astype(jnp.bfloat16), wgf[2][1],
                  c1w, c1b, c2w, c2b,
                  params["c3"]["W"], params["c3"]["b"][None, :])


# speedup vs baseline: 1.0817x; 1.0817x over previous
import functools
import numpy as np

import jax
import jax.numpy as jnp
from jax import lax
from jax.experimental import pallas as pl
from jax.experimental.pallas import tpu as pltpu
from jax.experimental.pallas import tpu_sc as plsc

EPS = 1e-5
NEG = -3.0e38
T = 256
W = 264
NGRAPH = 8
RN = 1000

_HI = jax.lax.Precision.HIGHEST
_dot = functools.partial(jnp.dot, preferred_element_type=jnp.float32,
                         precision=_HI)


def _dotb(a, b):
    return jax.lax.dot_general(
        a.astype(jnp.bfloat16), b,
        (((1,), (0,)), ((), ())), preferred_element_type=jnp.float32)


def _fold(lin, bn):
    inv = 1.0 / float(np.sqrt(1.0 + EPS))
    s = bn["gamma"] * inv
    return lin["W"] * s[None, :], (lin["b"] * s + bn["beta"])[None, :]


def _k3mm(v_ref, w_ref):
    acc = v_ref[:, 0:1] * w_ref[0:1, :]
    acc += v_ref[:, 1:2] * w_ref[1:2, :]
    acc += v_ref[:, 2:3] * w_ref[2:3, :]
    return acc


def _pre_body(x_ref, pos_ref, wx_ref, wp1_ref, wp2_ref, wp3_ref, b1_ref,
              a1_ref, bw1_ref, bw2_ref, bw3_ref):
    bw1 = _k3mm(pos_ref, wp1_ref)
    bw1_ref[...] = bw1
    bw2_ref[...] = _k3mm(pos_ref, wp2_ref)
    bw3_ref[...] = _k3mm(pos_ref, wp3_ref)
    a1_ref[...] = _k3mm(x_ref, wx_ref) + bw1 + b1_ref[...]


def _pre(x, pos, wx, wp1, wp2, wp3, b1):
    n = x.shape[0]
    nt = n // RN
    sh = jax.ShapeDtypeStruct((n, 64), jnp.float32)
    row = pl.BlockSpec((RN, 3), lambda i: (i, 0))
    w = pl.BlockSpec((3, 64), lambda i: (0, 0))
    return pl.pallas_call(
        _pre_body,
        grid=(nt,),
        in_specs=[row, row, w, w, w, w, pl.BlockSpec((1, 64), lambda i: (0, 0))],
        out_specs=tuple(pl.BlockSpec((RN, 64), lambda i: (i, 0))
                        for _ in range(4)),
        out_shape=(sh, sh, sh, sh),
    )(x, pos, wx, wp1, wp2, wp3, b1)


def _conv_body(n, e, cout, bases_ref, g_ref, dstr_ref, dstc_ref, bw_ref,
               w2_ref, b2_ref, w3_ref, b3_ref, out_ref):
    i = pl.program_id(0)

    @pl.when(i == 0)
    def _():
        out_ref[...] = jnp.full((n, cout), NEG, jnp.float32)

    base = pl.multiple_of(bases_ref[i], 8)
    d_row = dstr_ref[0]
    d_col = dstc_ref[0]
    j_row = d_row - base
    j_col = d_col - base

    win = bw_ref[pl.ds(base, W), :]
    iota_t_w = jax.lax.broadcasted_iota(jnp.int32, (T, W), 1)
    p2 = (j_col == iota_t_w).astype(jnp.bfloat16)
    bd = _dotb(p2, win)

    h = jnp.maximum(g_ref[:, :64] - bd, 0.0)
    h = jnp.maximum(_dotb(h, w2_ref[...]) + b2_ref[...], 0.0)
    h = _dotb(h, w3_ref[...]) + b3_ref[...]

    e0 = i * T
    valid = (jax.lax.broadcasted_iota(jnp.int32, (T, 1), 0) + e0) < e
    h = jnp.where(valid, h, NEG).astype(jnp.bfloat16)
    negb = jnp.asarray(NEG, jnp.bfloat16)

    s = 1
    while s < T:
        hs = jnp.concatenate([jnp.full((s, cout), negb, jnp.bfloat16),
                              h[:-s]], 0)
        ds = jnp.concatenate([jnp.full((s, 1), -1, jnp.int32), d_col[:-s]], 0)
        h = jnp.where(d_col == ds, jnp.maximum(h, hs), h)
        s *= 2

    nd = jnp.concatenate([d_row[:, 1:], jnp.full((1, 1), -1, jnp.int32)], 1)
    isend = d_row != nd
    iota_w_t = jax.lax.broadcasted_iota(jnp.int32, (W, T), 0)
    pend = ((iota_w_t == j_row) & isend).astype(jnp.bfloat16)
    outw = lax.dot_general(pend, h, (((1,), (0,)), ((), ())),
                           preferred_element_type=jnp.float32)
    hasend = jnp.max(pend.astype(jnp.float32), axis=1, keepdims=True)
    outw = jnp.where(hasend > 0.0, outw, NEG)

    cur = out_ref[pl.ds(base, W), :]
    out_ref[pl.ds(base, W), :] = jnp.maximum(cur, outw)


def _conv(bases, g, dstr, dstc, bw, w2, b2, w3, b3, n, e, cout):
    nt = dstr.shape[0]
    grid_spec = pltpu.PrefetchScalarGridSpec(
        num_scalar_prefetch=1,
        grid=(nt,),
        in_specs=[
            pl.BlockSpec((T, 128), lambda i, b: (i, 0)),
            pl.BlockSpec((1, 1, T), lambda i, b: (i, 0, 0)),
            pl.BlockSpec((1, T, 1), lambda i, b: (i, 0, 0)),
            pl.BlockSpec((n, 64), lambda i, b: (0, 0)),
            pl.BlockSpec((64, 128), lambda i, b: (0, 0)),
            pl.BlockSpec((1, 128), lambda i, b: (0, 0)),
            pl.BlockSpec((128, cout), lambda i, b: (0, 0)),
            pl.BlockSpec((1, cout), lambda i, b: (0, 0)),
        ],
        out_specs=pl.BlockSpec((n, cout), lambda i, b: (0, 0)),
    )
    return pl.pallas_call(
        functools.partial(_conv_body, n, e, cout),
        grid_spec=grid_spec,
        out_shape=jax.ShapeDtypeStruct((n, cout), jnp.float32),
    )(bases, g, dstr, dstc, bw, w2, b2, w3, b3)


def _post_body(agg_ref, wg_ref, bg_ref, wx_ref, bwn_ref, b1n_ref, a_ref):
    t = jnp.maximum(_dotb(agg_ref[...], wg_ref[...]) + bg_ref[...], 0.0)
    a_ref[...] = _dotb(t, wx_ref[...]) + bwn_ref[...] + b1n_ref[...]


def _post(agg, wg, bg, wx, bwn, b1n):
    n, cout = agg.shape
    nt = n // RN
    return pl.pallas_call(
        _post_body,
        grid=(nt,),
        in_specs=[
            pl.BlockSpec((RN, cout), lambda i: (i, 0)),
            pl.BlockSpec((cout, cout), lambda i: (0, 0)),
            pl.BlockSpec((1, cout), lambda i: (0, 0)),
            pl.BlockSpec((cout, 64), lambda i: (0, 0)),
            pl.BlockSpec((RN, 64), lambda i: (i, 0)),
            pl.BlockSpec((1, 64), lambda i: (0, 0)),
        ],
        out_specs=pl.BlockSpec((RN, 64), lambda i: (i, 0)),
        out_shape=jax.ShapeDtypeStruct((n, 64), jnp.float32),
    )(agg, wg, bg, wx, bwn, b1n)


def _final_body(nt, agg_ref, batch_ref, wg_ref, bg_ref, w1_ref, b1_ref,
                w2_ref, b2_ref, w3_ref, b3_ref, out_ref, gacc):
    i = pl.program_id(0)

    @pl.when(i == 0)
    def _():
        gacc[...] = jnp.full(gacc.shape, NEG, jnp.float32)

    h = jnp.maximum(_dotb(agg_ref[...], wg_ref[...]) + bg_ref[...], 0.0)
    b = batch_ref[0]
    for gidx in range(NGRAPH):
        m = b == gidx
        mx = jnp.max(jnp.where(m, h, NEG), axis=0, keepdims=True)
        gacc[gidx:gidx + 1, :] = jnp.maximum(gacc[gidx:gidx + 1, :], mx)

    @pl.when(i == nt - 1)
    def _():
        gv = gacc[...]
        gv = jnp.where(gv >= 0.0, gv, 0.0)
        z = jnp.maximum(_dot(gv, w1_ref[...]) + b1_ref[...], 0.0)
        z = jnp.maximum(_dot(z, w2_ref[...]) + b2_ref[...], 0.0)
        out_ref[...] = _dot(z, w3_ref[...]) + b3_ref[...]


def _final(agg, batch3, wg, bg, w1, b1, w2, b2, w3, b3):
    n, cout = agg.shape
    nt = n // RN
    return pl.pallas_call(
        functools.partial(_final_body, nt),
        grid=(nt,),
        in_specs=[
            pl.BlockSpec((RN, cout), lambda i: (i, 0)),
            pl.BlockSpec((1, RN, 1), lambda i: (i, 0, 0)),
            pl.BlockSpec((cout, cout), lambda i: (0, 0)),
            pl.BlockSpec((1, cout), lambda i: (0, 0)),
            pl.BlockSpec((cout, 256), lambda i: (0, 0)),
            pl.BlockSpec((1, 256), lambda i: (0, 0)),
            pl.BlockSpec((256, 128), lambda i: (0, 0)),
            pl.BlockSpec((1, 128), lambda i: (0, 0)),
            pl.BlockSpec((128, 7), lambda i: (0, 0)),
            pl.BlockSpec((1, 7), lambda i: (0, 0)),
        ],
        out_specs=pl.BlockSpec((NGRAPH, 7), lambda i: (0, 0)),
        out_shape=jax.ShapeDtypeStruct((NGRAPH, 7), jnp.float32),
        scratch_shapes=[pltpu.VMEM((NGRAPH, cout), jnp.float32)],
    )(agg, batch3, wg, bg, w1, b1, w2, b2, w3, b3)


NW = 32
SC_CHUNK = 280


def _sc_gather(table, idx_a):
    ep = idx_a.shape[0]
    b_per_w = ep // NW
    nchunk = b_per_w // SC_CHUNK
    d = table.shape[1]
    mesh = plsc.VectorSubcoreMesh(core_axis_name="c", subcore_axis_name="s")
    sh = jax.ShapeDtypeStruct((ep, d), jnp.float32)

    @functools.partial(
        pl.kernel, mesh=mesh,
        out_type=sh,
        scratch_types=[
            pltpu.VMEM((SC_CHUNK,), jnp.int32),
            pltpu.VMEM((SC_CHUNK, d), jnp.float32),
            pltpu.VMEM((SC_CHUNK,), jnp.int32),
            pltpu.VMEM((SC_CHUNK, d), jnp.float32),
            pltpu.SemaphoreType.DMA,
            pltpu.SemaphoreType.DMA,
        ],
    )
    def gk(table_hbm, ia_hbm, oa_hbm,
           idx0, rows0, idx1, rows1, sem0, sem1):
        wid = lax.axis_index("s") * 2 + lax.axis_index("c")
        base = wid * b_per_w
        bufs = [(idx0, rows0, sem0), (idx1, rows1, sem1)]
        work = [(ia_hbm, oa_hbm, c) for c in range(nchunk)]
        ih0, _, _ = work[0]
        pltpu.sync_copy(ih0.at[pl.ds(base, SC_CHUNK)], idx0)
        cp = pltpu.async_copy(table_hbm.at[idx0], rows0, sem0)
        for k in range(len(work)):
            if k + 1 < len(work):
                ihn, _, cn = work[k + 1]
                ivn, rvn, semn = bufs[(k + 1) % 2]
                pltpu.sync_copy(ihn.at[pl.ds(base + cn * SC_CHUNK,
                                             SC_CHUNK)], ivn)
                cpn = pltpu.async_copy(table_hbm.at[ivn], rvn, semn)
            cp.wait()
            _, oh, c = work[k]
            _, rv, _ = bufs[k % 2]
            pltpu.sync_copy(rv, oh.at[pl.ds(base + c * SC_CHUNK, SC_CHUNK)])
            if k + 1 < len(work):
                cp = cpn

    return gk(table, idx_a)


def kernel(x, pos, edge_index, batch, params):
    n = x.shape[0]
    e = edge_index.shape[1] + n
    nt_e = (e + T - 1) // T
    epad = nt_e * T

    sl = jnp.arange(n, dtype=edge_index.dtype)
    src = jnp.concatenate([edge_index[0], sl])
    dst = jnp.concatenate([edge_index[1], sl])
    order = jnp.argsort(dst)
    sdst = jnp.concatenate(
        [dst[order], jnp.full((epad - e,), n - 1, jnp.int32)])
    ssrc = jnp.concatenate([src[order], jnp.zeros((epad - e,), jnp.int32)])
    bases = jnp.minimum((sdst.reshape(nt_e, T)[:, 0] // 8) * 8, n - W)
    sdstr = sdst.reshape(nt_e, 1, T)
    sdstc = sdst.reshape(nt_e, T, 1)
    batch3 = batch.reshape(n // RN, RN, 1)

    cps = [params["conv1"], params["conv2"], params["conv3"]]
    w1f = [_fold(p["l1"], p["bn1"]) for p in cps]
    w2f = [_fold(p["l2"], p["bn2"]) for p in cps]
    wgf = [_fold(p["g"], p["bng"]) for p in cps]
    cins = [3, 128, 256]
    couts = [128, 256, 512]
    wx = [w1f[l][0][:cins[l]] for l in range(3)]
    wp = [w1f[l][0][cins[l]:] for l in range(3)]
    b1 = [w1f[l][1] for l in range(3)]

    a, bw1, bw2, bw3 = _pre(x, pos, wx[0], wp[0], wp[1], wp[2], b1[0])
    bws = [bw1, bw2, bw3]

    for l in range(3):
        tab = jnp.concatenate([a, bws[l]], axis=1)
        g = _sc_gather(tab, ssrc)
        agg = _conv(bases, g, sdstr, sdstc, bws[l],
                    w2f[l][0].astype(jnp.bfloat16), w2f[l][1],
                    cps[l]["l3"]["W"].astype(jnp.bfloat16),
                    cps[l]["l3"]["b"][None, :],
                    n, e, couts[l])
        if l < 2:
            a = _post(agg, wgf[l][0].astype(jnp.bfloat16), wgf[l][1],
                      wx[l + 1].astype(jnp.bfloat16), bws[l + 1], b1[l + 1])

    c1w, c1b = _fold(params["c1"], params["cbn1"])
    c2w, c2b = _fold(params["c2"], params["cbn2"])
    return _final(agg, batch3, wgf[2][0].astype(jnp.bfloat16), wgf[2][1],
                  c1w, c1b, c2w, c2b,
                  params["c3"]["W"], params["c3"]["b"][None, :])
